# baseline (speedup 1.0000x reference)
import functools

import jax
import jax.numpy as jnp
from jax import lax
from jax.experimental import pallas as pl
from jax.experimental.pallas import tpu as pltpu
from jax.experimental.pallas import tpu_sc as plsc

N = 10000
NP = 10240
D = 128
DH = 64
DO = 64
E = 320000
G = 16
DUMP = 10000

CK = 128
CHA = 160
CHD = 80
EP = 16 * CHA * CK
STRIPE = NP // 16


def _deg_body(dstr, vals, zeros, out, dst_v, vals_v, zbuf, deg_sh):
  c = lax.axis_index("c")
  s = lax.axis_index("s")
  wid = c * 16 + s
  pltpu.sync_copy(zeros, zbuf)
  for k in range(STRIPE // 64):
    pltpu.sync_copy(zbuf, deg_sh.at[pl.ds(s * STRIPE + k * 64, 64)])
  plsc.subcore_barrier()
  pltpu.sync_copy(dstr.at[wid], dst_v)
  pltpu.sync_copy(vals, vals_v)

  def body(j, carry):
    pltpu.sync_copy(vals_v, deg_sh.at[dst_v.at[j]], add=True)
    return carry

  lax.fori_loop(0, CHD, body, 0)
  plsc.subcore_barrier()
  pltpu.sync_copy(deg_sh.at[pl.ds(s * STRIPE, STRIPE)],
                  out.at[c, pl.ds(s * STRIPE, STRIPE)])


@functools.lru_cache(maxsize=None)
def _deg_call():
  mesh = plsc.VectorSubcoreMesh(core_axis_name="c", subcore_axis_name="s")
  return pl.kernel(
      _deg_body,
      out_type=jax.ShapeDtypeStruct((2, NP, 16), jnp.float32),
      mesh=mesh,
      scratch_types=[
          pltpu.VMEM((CHD, CK), jnp.int32),
          pltpu.VMEM((CK, 16), jnp.float32),
          pltpu.VMEM((64, 16), jnp.float32),
          pltpu.VMEM_SHARED((NP, 16), jnp.float32),
      ],
      compiler_params=pltpu.CompilerParams(use_tc_tiling_on_sc=False),
  )


NB = 4
LA = 2


def _agg_body(z2_hbm, srca, dstr, zeros, out, src_v, dst_v, rows, zbuf,
              agg_sh, gsem, ssem):
  c = lax.axis_index("c")
  s = lax.axis_index("s")
  pltpu.sync_copy(zeros, zbuf)
  for k in range(STRIPE // 64):
    pltpu.sync_copy(zbuf, agg_sh.at[pl.ds(s * STRIPE + k * 64, 64)])
  pltpu.sync_copy(srca.at[c, s], src_v)
  pltpu.sync_copy(dstr.at[s], dst_v)
  plsc.subcore_barrier()

  def gather(j, b):
    pltpu.async_copy(z2_hbm.at[src_v.at[j]], rows[b], gsem[b])

  def scatter(j, b):
    pltpu.async_copy(rows[b], agg_sh.at[dst_v.at[j]], ssem[b], add=True)

  def wait_gather(j, b):
    pltpu.make_async_copy(z2_hbm.at[src_v.at[j]], rows[b], gsem[b]).wait()

  def wait_scatter(j, b):
    pltpu.make_async_copy(rows[b], agg_sh.at[dst_v.at[j]], ssem[b]).wait()

  for b in range(LA):
    gather(b, b)

  def body(t, carry):
    for b in range(NB):
      j = t * NB + b
      wait_gather(j, b)
      scatter(j, b)
      bg = (b + LA) % NB
      jn = j + LA

      @pl.when(jn < CHA)
      def _next():
        @pl.when(jn >= NB)
        def _drain():
          wait_scatter(jn - NB, bg)
        gather(jn, bg)

    return carry

  lax.fori_loop(0, CHA // NB, body, 0)
  for b in range(NB):
    wait_scatter(CHA - NB + b, (CHA - NB + b) % NB)
  plsc.subcore_barrier()
  pltpu.sync_copy(agg_sh.at[pl.ds(s * STRIPE, STRIPE)],
                  out.at[c, pl.ds(s * STRIPE, STRIPE)])


@functools.lru_cache(maxsize=None)
def _agg_call():
  mesh = plsc.VectorSubcoreMesh(core_axis_name="c", subcore_axis_name="s")
  return pl.kernel(
      _agg_body,
      out_type=jax.ShapeDtypeStruct((2, NP, DH), jnp.float32),
      mesh=mesh,
      scratch_types=[
          pltpu.VMEM((CHA, CK), jnp.int32),
          pltpu.VMEM((CHA, CK), jnp.int32),
          [pltpu.VMEM((CK, DH), jnp.float32)] * NB,
          pltpu.VMEM((64, DH), jnp.float32),
          pltpu.VMEM_SHARED((NP, DH), jnp.float32),
          [pltpu.SemaphoreType.DMA] * NB,
          [pltpu.SemaphoreType.DMA] * NB,
      ],
      compiler_params=pltpu.CompilerParams(use_tc_tiling_on_sc=False),
  )


def _dinv_of(degp):
  deg = degp[0, :, 0:1] + degp[1, :, 0:1] + 1.0
  return lax.rsqrt(jnp.maximum(deg, 1.0))


def _split_store(z_ref, z):
  z_ref[0] = z[:, :DH]
  z_ref[1] = z[:, DH:]


def _tc1_body(x_ref, w_ref, degp_ref, z_ref):
  dinv = _dinv_of(degp_ref[...])
  _split_store(z_ref, jnp.dot(x_ref[...], w_ref[...],
                              preferred_element_type=jnp.float32) * dinv)


_tc1_call = pl.pallas_call(
    _tc1_body,
    grid=(NP // 1024,),
    in_specs=[
        pl.BlockSpec((1024, D), lambda i: (i, 0)),
        pl.BlockSpec((D, D), lambda i: (0, 0)),
        pl.BlockSpec((2, 1024, 16), lambda i: (0, i, 0)),
    ],
    out_specs=pl.BlockSpec((2, 1024, DH), lambda i: (0, i, 0)),
    out_shape=jax.ShapeDtypeStruct((2, NP, DH), jnp.float32),
)


def _tc2_body(s_ref, z_ref, degp_ref, b_ref, w_ref, z2_ref):
  dinv = _dinv_of(degp_ref[...])
  sp = s_ref[...]
  zp = z_ref[...]
  sz = jnp.concatenate([sp[0] + zp[0], sp[1] + zp[1]], axis=1)
  h = jax.nn.relu(dinv * sz + b_ref[...])
  _split_store(z2_ref, jnp.dot(h, w_ref[...],
                               preferred_element_type=jnp.float32) * dinv)


_tc2_call = pl.pallas_call(
    _tc2_body,
    grid=(NP // 1024,),
    in_specs=[
        pl.BlockSpec((2, 1024, DH), lambda i: (0, i, 0)),
        pl.BlockSpec((2, 1024, DH), lambda i: (0, i, 0)),
        pl.BlockSpec((2, 1024, 16), lambda i: (0, i, 0)),
        pl.BlockSpec((1, D), lambda i: (0, 0)),
        pl.BlockSpec((D, D), lambda i: (0, 0)),
    ],
    out_specs=pl.BlockSpec((2, 1024, DH), lambda i: (0, i, 0)),
    out_shape=jax.ShapeDtypeStruct((2, NP, DH), jnp.float32),
)


def _tc3_body(s_ref, z_ref, degp_ref, b_ref, batch_ref, wmu_ref, bmu_ref,
              wlv_ref, blv_ref, sums_ref, counts_ref, mu_ref, lv_ref):
  i = pl.program_id(0)
  dinv = _dinv_of(degp_ref[...])
  sp = s_ref[...]
  zp = z_ref[...]
  sz = jnp.concatenate([sp[0] + zp[0], sp[1] + zp[1]], axis=1)
  h = jax.nn.relu(dinv * sz + b_ref[...])
  onehot = (batch_ref[...] == lax.broadcasted_iota(jnp.int32, (1, G), 1)
            ).astype(jnp.float32)

  @pl.when(i == 0)
  def _init():
    sums_ref[...] = jnp.zeros_like(sums_ref)
    counts_ref[...] = jnp.zeros_like(counts_ref)

  sums_ref[...] += lax.dot_general(onehot, h, (((0,), (0,)), ((), ())),
                                   preferred_element_type=jnp.float32)
  counts_ref[...] += lax.dot_general(
      onehot, jnp.ones((1024, 1), jnp.float32), (((0,), (0,)), ((), ())),
      preferred_element_type=jnp.float32)

  @pl.when(i == pl.num_programs(0) - 1)
  def _fini():
    pooled = sums_ref[...] / jnp.maximum(counts_ref[...], 1.0)
    mu_ref[...] = jnp.dot(pooled, wmu_ref[...],
                          preferred_element_type=jnp.float32) + bmu_ref[...]
    lv_ref[...] = jnp.dot(pooled, wlv_ref[...],
                          preferred_element_type=jnp.float32) + blv_ref[...]


_tc3_call = pl.pallas_call(
    _tc3_body,
    grid=(NP // 1024,),
    in_specs=[
        pl.BlockSpec((2, 1024, DH), lambda i: (0, i, 0)),
        pl.BlockSpec((2, 1024, DH), lambda i: (0, i, 0)),
        pl.BlockSpec((2, 1024, 16), lambda i: (0, i, 0)),
        pl.BlockSpec((1, D), lambda i: (0, 0)),
        pl.BlockSpec((1024, 1), lambda i: (i, 0)),
        pl.BlockSpec((D, DO), lambda i: (0, 0)),
        pl.BlockSpec((1, DO), lambda i: (0, 0)),
        pl.BlockSpec((D, DO), lambda i: (0, 0)),
        pl.BlockSpec((1, DO), lambda i: (0, 0)),
    ],
    out_specs=[
        pl.BlockSpec((G, D), lambda i: (0, 0)),
        pl.BlockSpec((G, 1), lambda i: (0, 0)),
        pl.BlockSpec((G, DO), lambda i: (0, 0)),
        pl.BlockSpec((G, DO), lambda i: (0, 0)),
    ],
    out_shape=[
        jax.ShapeDtypeStruct((G, D), jnp.float32),
        jax.ShapeDtypeStruct((G, 1), jnp.float32),
        jax.ShapeDtypeStruct((G, DO), jnp.float32),
        jax.ShapeDtypeStruct((G, DO), jnp.float32),
    ],
)


@jax.jit
def _run(x, edge_index, batch, W1, b1, W2, b2, Wmu, bmu, Wlv, blv):
  src = edge_index[0]
  dst = edge_index[1]
  pad = jnp.full((EP - E,), DUMP, dtype=jnp.int32)
  src_p = jnp.concatenate([src, pad])
  dst_p = jnp.concatenate([dst, pad])
  srca = jnp.stack([src_p, src_p + NP]).reshape(2, 16, CHA, CK)
  dsta = dst_p.reshape(16, CHA, CK)
  dstd = dst_p.reshape(32, CHD, CK)

  x_p = jnp.pad(x, ((0, NP - N), (0, 0)))
  batch_p = jnp.pad(batch, (0, NP - N), constant_values=G).reshape(NP, 1)

  zeros_h = jnp.zeros((64, DH), jnp.float32)
  zeros16 = jnp.zeros((64, 16), jnp.float32)
  ones16 = jnp.zeros((CK, 16), jnp.float32).at[:, 0].set(1.0)

  degp = _deg_call()(dstd, ones16, zeros16)
  z1 = _tc1_call(x_p, W1, degp)
  s1 = _agg_call()(z1.reshape(2 * NP, DH), srca, dsta, zeros_h)
  z2 = _tc2_call(s1, z1, degp, b1.reshape(1, D), W2)
  s2 = _agg_call()(z2.reshape(2 * NP, DH), srca, dsta, zeros_h)
  _, _, mu, logvar = _tc3_call(s2, z2, degp, b2.reshape(1, D), batch_p,
                               Wmu, bmu.reshape(1, DO), Wlv, blv.reshape(1, DO))
  return mu, logvar


def kernel(x, edge_index, batch, W1, b1, W2, b2, Wmu, bmu, Wlv, blv):
  return _run(x, edge_index, batch, W1, b1, W2, b2, Wmu, bmu, Wlv, blv)


# speedup vs baseline: 1.5088x; 1.5088x over previous
import functools

import jax
import jax.numpy as jnp
from jax import lax
from jax.experimental import pallas as pl
from jax.experimental.pallas import tpu as pltpu
from jax.experimental.pallas import tpu_sc as plsc

N = 10000
NP = 10240
D = 128
DH = 64
DO = 64
E = 320000
G = 16
DUMP = 10000

CK = 128
CHA = 160
CHD = 80
EP = 16 * CHA * CK
STRIPE = NP // 16


def _deg_body(dstr, vals, zeros, out, dst_v, vals_v, zbuf, deg_sh):
  c = lax.axis_index("c")
  s = lax.axis_index("s")
  wid = c * 16 + s
  pltpu.sync_copy(zeros, zbuf)
  for k in range(STRIPE // 64):
    pltpu.sync_copy(zbuf, deg_sh.at[pl.ds(s * STRIPE + k * 64, 64)])
  plsc.subcore_barrier()
  pltpu.sync_copy(dstr.at[wid], dst_v)
  pltpu.sync_copy(vals, vals_v)

  def body(j, carry):
    pltpu.sync_copy(vals_v, deg_sh.at[dst_v.at[j]], add=True)
    return carry

  lax.fori_loop(0, CHD, body, 0)
  plsc.subcore_barrier()
  pltpu.sync_copy(deg_sh.at[pl.ds(s * STRIPE, STRIPE)],
                  out.at[c, pl.ds(s * STRIPE, STRIPE)])


@functools.lru_cache(maxsize=None)
def _deg_call():
  mesh = plsc.VectorSubcoreMesh(core_axis_name="c", subcore_axis_name="s")
  return pl.kernel(
      _deg_body,
      out_type=jax.ShapeDtypeStruct((2, NP, 16), jnp.float32),
      mesh=mesh,
      scratch_types=[
          pltpu.VMEM((CHD, CK), jnp.int32),
          pltpu.VMEM((CK, 16), jnp.float32),
          pltpu.VMEM((64, 16), jnp.float32),
          pltpu.VMEM_SHARED((NP, 16), jnp.float32),
      ],
      compiler_params=pltpu.CompilerParams(use_tc_tiling_on_sc=False),
  )


NB = 4
LA = 2


def _agg_body(z2_hbm, srca, dstr, zeros, out, src_v, dst_v, rows, zbuf,
              agg_sh, gsem, ssem):
  c = lax.axis_index("c")
  s = lax.axis_index("s")
  pltpu.sync_copy(zeros, zbuf)
  for k in range(STRIPE // 64):
    pltpu.sync_copy(zbuf, agg_sh.at[pl.ds(s * STRIPE + k * 64, 64)])
  pltpu.sync_copy(srca.at[c, s], src_v)
  pltpu.sync_copy(dstr.at[s], dst_v)
  plsc.subcore_barrier()

  def gather(j, b):
    pltpu.async_copy(z2_hbm.at[src_v.at[j]], rows[b], gsem[b])

  def scatter(j, b):
    pltpu.async_copy(rows[b], agg_sh.at[dst_v.at[j]], ssem[b], add=True)

  def wait_gather(j, b):
    pltpu.make_async_copy(z2_hbm.at[src_v.at[j]], rows[b], gsem[b]).wait()

  def wait_scatter(j, b):
    pltpu.make_async_copy(rows[b], agg_sh.at[dst_v.at[j]], ssem[b]).wait()

  for b in range(LA):
    gather(b, b)

  def body(t, carry):
    for b in range(NB):
      j = t * NB + b
      wait_gather(j, b)
      scatter(j, b)
      bg = (b + LA) % NB
      jn = j + LA

      @pl.when(jn < CHA)
      def _next():
        @pl.when(jn >= NB)
        def _drain():
          wait_scatter(jn - NB, bg)
        gather(jn, bg)

    return carry

  lax.fori_loop(0, CHA // NB, body, 0)
  for b in range(NB):
    wait_scatter(CHA - NB + b, (CHA - NB + b) % NB)
  plsc.subcore_barrier()
  pltpu.sync_copy(agg_sh.at[pl.ds(s * STRIPE, STRIPE)],
                  out.at[c, pl.ds(s * STRIPE, STRIPE)])


@functools.lru_cache(maxsize=None)
def _agg_call():
  mesh = plsc.VectorSubcoreMesh(core_axis_name="c", subcore_axis_name="s")
  return pl.kernel(
      _agg_body,
      out_type=jax.ShapeDtypeStruct((2, NP, DH), jnp.bfloat16),
      mesh=mesh,
      scratch_types=[
          pltpu.VMEM((CHA, CK), jnp.int32),
          pltpu.VMEM((CHA, CK), jnp.int32),
          [pltpu.VMEM((CK, DH), jnp.bfloat16)] * NB,
          pltpu.VMEM((64, DH), jnp.bfloat16),
          pltpu.VMEM_SHARED((NP, DH), jnp.bfloat16),
          [pltpu.SemaphoreType.DMA] * NB,
          [pltpu.SemaphoreType.DMA] * NB,
      ],
      compiler_params=pltpu.CompilerParams(use_tc_tiling_on_sc=False),
  )


def _dinv_of(degp):
  deg = degp[0, :, 0:1] + degp[1, :, 0:1] + 1.0
  return lax.rsqrt(jnp.maximum(deg, 1.0))


def _split_store(z_ref, z):
  zh = z.astype(jnp.bfloat16)
  z_ref[0] = zh[:, :DH]
  z_ref[1] = zh[:, DH:]


def _tc1_body(x_ref, w_ref, degp_ref, z_ref):
  dinv = _dinv_of(degp_ref[...])
  _split_store(z_ref, jnp.dot(x_ref[...], w_ref[...],
                              preferred_element_type=jnp.float32) * dinv)


_tc1_call = pl.pallas_call(
    _tc1_body,
    grid=(NP // 1024,),
    in_specs=[
        pl.BlockSpec((1024, D), lambda i: (i, 0)),
        pl.BlockSpec((D, D), lambda i: (0, 0)),
        pl.BlockSpec((2, 1024, 16), lambda i: (0, i, 0)),
    ],
    out_specs=pl.BlockSpec((2, 1024, DH), lambda i: (0, i, 0)),
    out_shape=jax.ShapeDtypeStruct((2, NP, DH), jnp.bfloat16),
)


def _tc2_body(s_ref, z_ref, degp_ref, b_ref, w_ref, z2_ref):
  dinv = _dinv_of(degp_ref[...])
  sp = s_ref[...].astype(jnp.float32)
  zp = z_ref[...].astype(jnp.float32)
  sz = jnp.concatenate([sp[0] + zp[0], sp[1] + zp[1]], axis=1)
  h = jax.nn.relu(dinv * sz + b_ref[...])
  _split_store(z2_ref, jnp.dot(h, w_ref[...],
                               preferred_element_type=jnp.float32) * dinv)


_tc2_call = pl.pallas_call(
    _tc2_body,
    grid=(NP // 1024,),
    in_specs=[
        pl.BlockSpec((2, 1024, DH), lambda i: (0, i, 0)),
        pl.BlockSpec((2, 1024, DH), lambda i: (0, i, 0)),
        pl.BlockSpec((2, 1024, 16), lambda i: (0, i, 0)),
        pl.BlockSpec((1, D), lambda i: (0, 0)),
        pl.BlockSpec((D, D), lambda i: (0, 0)),
    ],
    out_specs=pl.BlockSpec((2, 1024, DH), lambda i: (0, i, 0)),
    out_shape=jax.ShapeDtypeStruct((2, NP, DH), jnp.bfloat16),
)


def _tc3_body(s_ref, z_ref, degp_ref, b_ref, batch_ref, wmu_ref, bmu_ref,
              wlv_ref, blv_ref, sums_ref, counts_ref, mu_ref, lv_ref):
  i = pl.program_id(0)
  dinv = _dinv_of(degp_ref[...])
  sp = s_ref[...].astype(jnp.float32)
  zp = z_ref[...].astype(jnp.float32)
  sz = jnp.concatenate([sp[0] + zp[0], sp[1] + zp[1]], axis=1)
  h = jax.nn.relu(dinv * sz + b_ref[...])
  onehot = (batch_ref[...] == lax.broadcasted_iota(jnp.int32, (1, G), 1)
            ).astype(jnp.float32)

  @pl.when(i == 0)
  def _init():
    sums_ref[...] = jnp.zeros_like(sums_ref)
    counts_ref[...] = jnp.zeros_like(counts_ref)

  sums_ref[...] += lax.dot_general(onehot, h, (((0,), (0,)), ((), ())),
                                   preferred_element_type=jnp.float32)
  counts_ref[...] += lax.dot_general(
      onehot, jnp.ones((1024, 1), jnp.float32), (((0,), (0,)), ((), ())),
      preferred_element_type=jnp.float32)

  @pl.when(i == pl.num_programs(0) - 1)
  def _fini():
    pooled = sums_ref[...] / jnp.maximum(counts_ref[...], 1.0)
    mu_ref[...] = jnp.dot(pooled, wmu_ref[...],
                          preferred_element_type=jnp.float32) + bmu_ref[...]
    lv_ref[...] = jnp.dot(pooled, wlv_ref[...],
                          preferred_element_type=jnp.float32) + blv_ref[...]


_tc3_call = pl.pallas_call(
    _tc3_body,
    grid=(NP // 1024,),
    in_specs=[
        pl.BlockSpec((2, 1024, DH), lambda i: (0, i, 0)),
        pl.BlockSpec((2, 1024, DH), lambda i: (0, i, 0)),
        pl.BlockSpec((2, 1024, 16), lambda i: (0, i, 0)),
        pl.BlockSpec((1, D), lambda i: (0, 0)),
        pl.BlockSpec((1024, 1), lambda i: (i, 0)),
        pl.BlockSpec((D, DO), lambda i: (0, 0)),
        pl.BlockSpec((1, DO), lambda i: (0, 0)),
        pl.BlockSpec((D, DO), lambda i: (0, 0)),
        pl.BlockSpec((1, DO), lambda i: (0, 0)),
    ],
    out_specs=[
        pl.BlockSpec((G, D), lambda i: (0, 0)),
        pl.BlockSpec((G, 1), lambda i: (0, 0)),
        pl.BlockSpec((G, DO), lambda i: (0, 0)),
        pl.BlockSpec((G, DO), lambda i: (0, 0)),
    ],
    out_shape=[
        jax.ShapeDtypeStruct((G, D), jnp.float32),
        jax.ShapeDtypeStruct((G, 1), jnp.float32),
        jax.ShapeDtypeStruct((G, DO), jnp.float32),
        jax.ShapeDtypeStruct((G, DO), jnp.float32),
    ],
)


@jax.jit
def _run(x, edge_index, batch, W1, b1, W2, b2, Wmu, bmu, Wlv, blv):
  src = edge_index[0]
  dst = edge_index[1]
  pad = jnp.full((EP - E,), DUMP, dtype=jnp.int32)
  src_p = jnp.concatenate([src, pad])
  dst_p = jnp.concatenate([dst, pad])
  srca = jnp.stack([src_p, src_p + NP]).reshape(2, 16, CHA, CK)
  dsta = dst_p.reshape(16, CHA, CK)
  dstd = dst_p.reshape(32, CHD, CK)

  x_p = jnp.pad(x, ((0, NP - N), (0, 0)))
  batch_p = jnp.pad(batch, (0, NP - N), constant_values=G).reshape(NP, 1)

  zeros_h = jnp.zeros((64, DH), jnp.bfloat16)
  zeros16 = jnp.zeros((64, 16), jnp.float32)
  ones16 = jnp.zeros((CK, 16), jnp.float32).at[:, 0].set(1.0)

  degp = _deg_call()(dstd, ones16, zeros16)
  z1 = _tc1_call(x_p, W1, degp)
  s1 = _agg_call()(z1.reshape(2 * NP, DH), srca, dsta, zeros_h)
  z2 = _tc2_call(s1, z1, degp, b1.reshape(1, D), W2)
  s2 = _agg_call()(z2.reshape(2 * NP, DH), srca, dsta, zeros_h)
  _, _, mu, logvar = _tc3_call(s2, z2, degp, b2.reshape(1, D), batch_p,
                               Wmu, bmu.reshape(1, DO), Wlv, blv.reshape(1, DO))
  return mu, logvar


def kernel(x, edge_index, batch, W1, b1, W2, b2, Wmu, bmu, Wlv, blv):
  return _run(x, edge_index, batch, W1, b1, W2, b2, Wmu, bmu, Wlv, blv)
